# baseline (speedup 1.0000x reference)
import functools

import jax
import jax.numpy as jnp
from jax import lax
from jax.experimental import pallas as pl
from jax.experimental.pallas import tpu as pltpu
from jax.experimental.pallas import tpu_sc as plsc

B = 16384
D = 128
L = 16
NC, NS = 2, 16
NW = NC * NS
BPW = B // NW
CHUNK = 64
NCHUNK = BPW // CHUNK
G = CHUNK // L
RING = 4
PRIME = RING - 1


def _gmf_body(uv_hbm, iv_hbm, ut_hbm, it_hbm, w_hbm, b_hbm, out_hbm,
              uidx_v, iidx_v, w_v, b_v, ubufs, ibufs, obufs,
              sau, sai, saw, sab, sus, sis, sos):
    wid = lax.axis_index("s") * NC + lax.axis_index("c")
    base = wid * BPW

    cu0 = pltpu.make_async_copy(uv_hbm.at[pl.ds(base, BPW)], uidx_v, sau)
    ci0 = pltpu.make_async_copy(iv_hbm.at[pl.ds(base, BPW)], iidx_v, sai)
    cu0.start()
    ci0.start()

    def start(c, slot):
        off = pl.multiple_of(c * CHUNK, CHUNK)
        pltpu.make_async_copy(
            ut_hbm.at[uidx_v.at[pl.ds(off, CHUNK)]], ubufs[slot],
            sus[slot]).start()
        pltpu.make_async_copy(
            it_hbm.at[iidx_v.at[pl.ds(off, CHUNK)]], ibufs[slot],
            sis[slot]).start()

    cw = pltpu.make_async_copy(w_hbm, w_v, saw)
    cb = pltpu.make_async_copy(b_hbm, b_v, sab)
    cw.start()
    cb.start()
    cu0.wait()
    ci0.wait()
    for s in range(PRIME):
        start(s, s)
    cw.wait()
    cb.wait()

    row0 = lax.broadcasted_iota(jnp.int32, (L,), 0)
    row_ids = [row0 + g * L for g in range(G)]
    zeros = tuple(jnp.zeros((L,), jnp.float32) for _ in range(G))
    bvec = plsc.load_gather(b_v, [jnp.zeros((L,), jnp.int32)])

    @pl.loop(0, NCHUNK, step=RING)
    def _chunk_quad(c0):
        for slot in range(RING):
            c = c0 + slot
            ub, ib, ob = ubufs[slot], ibufs[slot], obufs[slot]

            @pl.when(c + PRIME < NCHUNK)
            def _():
                start(c + PRIME, (slot + PRIME) % RING)

            pltpu.make_async_copy(
                ut_hbm.at[uidx_v.at[pl.ds(0, CHUNK)]], ub, sus[slot]).wait()
            pltpu.make_async_copy(
                it_hbm.at[uidx_v.at[pl.ds(0, CHUNK)]], ib, sis[slot]).wait()

            def dbody(d, accs, ub=ub, ib=ib):
                col = (row0 + d) & (D - 1)
                wb = plsc.load_gather(w_v, [col])
                new = []
                for g in range(G):
                    uvals = plsc.load_gather(ub, [row_ids[g], col])
                    ivals = plsc.load_gather(ib, [row_ids[g], col])
                    new.append(accs[g] + uvals * ivals * wb)
                return tuple(new)

            accs = plsc.parallel_loop(0, D, unroll=2, carry=zeros)(dbody)

            @pl.when(c >= RING)
            def _():
                pltpu.make_async_copy(
                    ob, out_hbm.at[pl.ds(0, CHUNK)], sos[slot]).wait()

            for g in range(G):
                logits = accs[g] + bvec
                ob[pl.ds(g * L, L)] = 1.0 / (1.0 + jnp.exp(-logits))
            pltpu.make_async_copy(
                ob, out_hbm.at[pl.ds(pl.multiple_of(base + c * CHUNK, CHUNK),
                                     CHUNK)], sos[slot]).start()

    for s in range(RING):
        pltpu.make_async_copy(
            obufs[s], out_hbm.at[pl.ds(0, CHUNK)], sos[s]).wait()


@jax.jit
def kernel(user_vector, item_vector, user_table, item_table, h_w, h_b):
    mesh = plsc.VectorSubcoreMesh(core_axis_name="c", subcore_axis_name="s",
                                  num_cores=NC, num_subcores=NS)
    k = pl.kernel(
        _gmf_body,
        out_type=jax.ShapeDtypeStruct((B,), jnp.float32),
        mesh=mesh,
        scratch_types=[
            pltpu.VMEM((BPW,), jnp.int32),
            pltpu.VMEM((BPW,), jnp.int32),
            pltpu.VMEM((D,), jnp.float32),
            pltpu.VMEM((1,), jnp.float32),
            [pltpu.VMEM((CHUNK, D), jnp.float32) for _ in range(RING)],
            [pltpu.VMEM((CHUNK, D), jnp.float32) for _ in range(RING)],
            [pltpu.VMEM((CHUNK,), jnp.float32) for _ in range(RING)],
            pltpu.SemaphoreType.DMA,
            pltpu.SemaphoreType.DMA,
            pltpu.SemaphoreType.DMA,
            pltpu.SemaphoreType.DMA,
            [pltpu.SemaphoreType.DMA for _ in range(RING)],
            [pltpu.SemaphoreType.DMA for _ in range(RING)],
            [pltpu.SemaphoreType.DMA for _ in range(RING)],
        ],
        compiler_params=pltpu.CompilerParams(needs_layout_passes=False),
        name="gmf_sc",
    )
    out = k(user_vector.astype(jnp.int32), item_vector.astype(jnp.int32),
            user_table, item_table,
            h_w.reshape(D).astype(jnp.float32),
            h_b.astype(jnp.float32).reshape(1))
    return out.reshape(B, 1)


# speedup vs baseline: 1.0076x; 1.0076x over previous
import functools

import jax
import jax.numpy as jnp
from jax import lax
from jax.experimental import pallas as pl
from jax.experimental.pallas import tpu as pltpu
from jax.experimental.pallas import tpu_sc as plsc

B = 16384
D = 128
L = 16
NC, NS = 2, 16
NW = NC * NS
BPW = B // NW
CHUNK = 64
NCHUNK = BPW // CHUNK
G = CHUNK // L
RING = 4
PRIME = RING - 1


def _gmf_body(uv_hbm, iv_hbm, ut_hbm, it_hbm, w_hbm, b_hbm, out_hbm,
              uidx_v, iidx_v, w_v, b_v, ubufs, ibufs, obufs,
              sau, sai, saw, sab, sus, sis, sos):
    wid = lax.axis_index("s") * NC + lax.axis_index("c")
    base = wid * BPW

    cu0 = pltpu.make_async_copy(uv_hbm.at[pl.ds(base, BPW)], uidx_v, sau)
    ci0 = pltpu.make_async_copy(iv_hbm.at[pl.ds(base, BPW)], iidx_v, sai)
    cu0.start()
    ci0.start()

    def start(c, slot):
        off = pl.multiple_of(c * CHUNK, CHUNK)
        pltpu.make_async_copy(
            ut_hbm.at[uidx_v.at[pl.ds(off, CHUNK)]], ubufs[slot],
            sus[slot]).start()
        pltpu.make_async_copy(
            it_hbm.at[iidx_v.at[pl.ds(off, CHUNK)]], ibufs[slot],
            sis[slot]).start()

    cw = pltpu.make_async_copy(w_hbm, w_v, saw)
    cb = pltpu.make_async_copy(b_hbm, b_v, sab)
    cw.start()
    cb.start()
    cu0.wait()
    ci0.wait()
    for s in range(PRIME):
        start(s, s)
    cw.wait()
    cb.wait()

    row0 = lax.broadcasted_iota(jnp.int32, (L,), 0)
    row_ids = [row0 + g * L for g in range(G)]
    zeros = tuple(jnp.zeros((L,), jnp.float32) for _ in range(G))
    bvec = plsc.load_gather(b_v, [jnp.zeros((L,), jnp.int32)])

    @pl.loop(0, NCHUNK, step=RING)
    def _chunk_quad(c0):
        for slot in range(RING):
            c = c0 + slot
            ub, ib, ob = ubufs[slot], ibufs[slot], obufs[slot]

            @pl.when(c + PRIME < NCHUNK)
            def _():
                start(c + PRIME, (slot + PRIME) % RING)

            pltpu.make_async_copy(
                ut_hbm.at[uidx_v.at[pl.ds(0, CHUNK)]], ub, sus[slot]).wait()
            pltpu.make_async_copy(
                it_hbm.at[uidx_v.at[pl.ds(0, CHUNK)]], ib, sis[slot]).wait()

            def dbody(d, accs, ub=ub, ib=ib):
                col = (row0 + d) & (D - 1)
                wb = plsc.load_gather(w_v, [col])
                new = []
                for g in range(G):
                    uvals = plsc.load_gather(ub, [row_ids[g], col])
                    ivals = plsc.load_gather(ib, [row_ids[g], col])
                    new.append(accs[g] + uvals * ivals * wb)
                return tuple(new)

            accs = plsc.parallel_loop(0, D, carry=zeros)(dbody)

            @pl.when(c >= RING)
            def _():
                pltpu.make_async_copy(
                    ob, out_hbm.at[pl.ds(0, CHUNK)], sos[slot]).wait()

            for g in range(G):
                logits = accs[g] + bvec
                ob[pl.ds(g * L, L)] = 1.0 / (1.0 + jnp.exp(-logits))
            pltpu.make_async_copy(
                ob, out_hbm.at[pl.ds(pl.multiple_of(base + c * CHUNK, CHUNK),
                                     CHUNK)], sos[slot]).start()

    for s in range(RING):
        pltpu.make_async_copy(
            obufs[s], out_hbm.at[pl.ds(0, CHUNK)], sos[s]).wait()


@jax.jit
def kernel(user_vector, item_vector, user_table, item_table, h_w, h_b):
    mesh = plsc.VectorSubcoreMesh(core_axis_name="c", subcore_axis_name="s",
                                  num_cores=NC, num_subcores=NS)
    k = pl.kernel(
        _gmf_body,
        out_type=jax.ShapeDtypeStruct((B,), jnp.float32),
        mesh=mesh,
        scratch_types=[
            pltpu.VMEM((BPW,), jnp.int32),
            pltpu.VMEM((BPW,), jnp.int32),
            pltpu.VMEM((D,), jnp.float32),
            pltpu.VMEM((1,), jnp.float32),
            [pltpu.VMEM((CHUNK, D), jnp.float32) for _ in range(RING)],
            [pltpu.VMEM((CHUNK, D), jnp.float32) for _ in range(RING)],
            [pltpu.VMEM((CHUNK,), jnp.float32) for _ in range(RING)],
            pltpu.SemaphoreType.DMA,
            pltpu.SemaphoreType.DMA,
            pltpu.SemaphoreType.DMA,
            pltpu.SemaphoreType.DMA,
            [pltpu.SemaphoreType.DMA for _ in range(RING)],
            [pltpu.SemaphoreType.DMA for _ in range(RING)],
            [pltpu.SemaphoreType.DMA for _ in range(RING)],
        ],
        compiler_params=pltpu.CompilerParams(needs_layout_passes=False),
        name="gmf_sc",
    )
    out = k(user_vector.astype(jnp.int32), item_vector.astype(jnp.int32),
            user_table, item_table,
            h_w.reshape(D).astype(jnp.float32),
            h_b.astype(jnp.float32).reshape(1))
    return out.reshape(B, 1)
